# Initial kernel scaffold; baseline (speedup 1.0000x reference)
import jax
import jax.numpy as jnp
from jax.experimental import pallas as pl


def kernel(x, pe_table):
    raise NotImplementedError("write your pallas kernel here")


# speedup vs baseline: 1.6615x; 1.6615x over previous
import jax
import jax.numpy as jnp
from jax.experimental import pallas as pl


_BLK = 512


def _add_pe_kernel(x_ref, pe_ref, o_ref):
    o_ref[0] = x_ref[0] + pe_ref[...]


def kernel(x, pe_table):
    batch, seq_len, d_model = x.shape
    blk = min(_BLK, seq_len)
    grid = (seq_len // blk, batch)
    return pl.pallas_call(
        _add_pe_kernel,
        grid=grid,
        in_specs=[
            pl.BlockSpec((1, blk, d_model), lambda s, b: (b, s, 0)),
            pl.BlockSpec((blk, d_model), lambda s, b: (s, 0)),
        ],
        out_specs=pl.BlockSpec((1, blk, d_model), lambda s, b: (b, s, 0)),
        out_shape=jax.ShapeDtypeStruct(x.shape, x.dtype),
    )(x, pe_table[:seq_len])


# speedup vs baseline: 1.7144x; 1.0318x over previous
import jax
import jax.numpy as jnp
from jax.experimental import pallas as pl


_BLK = 256


def _add_pe_kernel(x_ref, pe_ref, o_ref):
    o_ref[...] = x_ref[...] + pe_ref[...][None, :, :]


def kernel(x, pe_table):
    batch, seq_len, d_model = x.shape
    blk = min(_BLK, seq_len)
    grid = (seq_len // blk,)
    return pl.pallas_call(
        _add_pe_kernel,
        grid=grid,
        in_specs=[
            pl.BlockSpec((batch, blk, d_model), lambda s: (0, s, 0)),
            pl.BlockSpec((blk, d_model), lambda s: (s, 0)),
        ],
        out_specs=pl.BlockSpec((batch, blk, d_model), lambda s: (0, s, 0)),
        out_shape=jax.ShapeDtypeStruct(x.shape, x.dtype),
    )(x, pe_table[:seq_len])
